# baseline (speedup 1.0000x reference)
import jax
import jax.numpy as jnp
from jax import lax
from jax.experimental import pallas as pl
from jax.experimental.pallas import tpu as pltpu

B, H, W = 8192, 64, 64
BLOCK_B = 512
NUM_BLOCKS = B // BLOCK_B


def _loss_block_kernel(x_ref, out_ref):
    xb = x_ref[...]
    m = jnp.max(xb, axis=(1, 2), keepdims=True)
    j3 = lax.broadcasted_iota(jnp.int32, xb.shape, 1)
    k3 = lax.broadcasted_iota(jnp.int32, xb.shape, 2)
    flat = j3 * W + k3
    idx = jnp.min(jnp.where(xb == m, flat, H * W), axis=(1, 2), keepdims=True)
    mx = (idx // W).astype(jnp.float32)
    my = (idx % W).astype(jnp.float32)
    jf = j3.astype(jnp.float32)
    kf = k3.astype(jnp.float32)
    dist = (mx - jf) ** 2 + (my - kf) ** 2
    partial = jnp.sum(dist * xb)
    out_ref[...] = jnp.full((1, 1, 128), partial, dtype=jnp.float32)


def kernel(x):
    partials = pl.pallas_call(
        _loss_block_kernel,
        grid=(NUM_BLOCKS,),
        in_specs=[
            pl.BlockSpec((BLOCK_B, H, W), lambda i: (i, 0, 0)),
        ],
        out_specs=pl.BlockSpec((1, 1, 128), lambda i: (i, 0, 0)),
        out_shape=jax.ShapeDtypeStruct((NUM_BLOCKS, 1, 128), jnp.float32),
        compiler_params=pltpu.CompilerParams(
            dimension_semantics=("parallel",),
        ),
    )(x)
    return jnp.sum(partials[:, 0, 0]).reshape(1)


# speedup vs baseline: 2.1844x; 2.1844x over previous
import jax
import jax.numpy as jnp
from jax import lax
from jax.experimental import pallas as pl
from jax.experimental.pallas import tpu as pltpu

B, H, W = 8192, 64, 64
HW = H * W
BLOCK_B = 512
NUM_BLOCKS = B // BLOCK_B


def _loss_block_kernel(x_ref, out_ref):
    xb = x_ref[...]
    m = jnp.max(xb, axis=1, keepdims=True)
    p = lax.broadcasted_iota(jnp.int32, (1, HW), 1)
    idx = jnp.min(jnp.where(xb == m, p, HW), axis=1, keepdims=True)
    mx = (idx // W).astype(jnp.float32)
    my = (idx % W).astype(jnp.float32)
    jf = (p // W).astype(jnp.float32)
    kf = (p % W).astype(jnp.float32)
    s0 = jnp.sum(xb, axis=1, keepdims=True)
    sj = jnp.sum(xb * jf, axis=1, keepdims=True)
    sk = jnp.sum(xb * kf, axis=1, keepdims=True)
    cs = jnp.sum(xb, axis=0, keepdims=True)
    c2 = jf * jf + kf * kf
    s2g = jnp.sum(cs * c2)
    loss_b = (mx * mx + my * my) * s0 - 2.0 * (mx * sj + my * sk)
    out_ref[...] = jnp.full((1, 1, 128), jnp.sum(loss_b) + s2g,
                            dtype=jnp.float32)


def kernel(x):
    x2 = x.reshape(B, HW)
    partials = pl.pallas_call(
        _loss_block_kernel,
        grid=(NUM_BLOCKS,),
        in_specs=[
            pl.BlockSpec((BLOCK_B, HW), lambda i: (i, 0)),
        ],
        out_specs=pl.BlockSpec((1, 1, 128), lambda i: (i, 0, 0)),
        out_shape=jax.ShapeDtypeStruct((NUM_BLOCKS, 1, 128), jnp.float32),
        compiler_params=pltpu.CompilerParams(
            dimension_semantics=("parallel",),
        ),
    )(x2)
    return jnp.sum(partials[:, 0, 0]).reshape(1)
